# baseline (speedup 1.0000x reference)
import functools

import jax
import jax.numpy as jnp
from jax import lax
from jax.experimental import pallas as pl
from jax.experimental.pallas import tpu as pltpu
from jax.experimental.pallas import tpu_sc as plsc

_N = 1025
_N2 = _N * _N
_H = 16
_V = 3972
_L = 16
_GROUPS = (_N + _L - 1) // _L
_NRG = 8
_NHG = 4
_HPW = _H // _NHG
_RPB = 8
_STEPS = 1024 // (_NRG * _RPB)
_IDXBUF = _RPB * _N + 2 * _L
_OBLK = 8224
_OBUF = _HPW * _OBLK


def _body(table_hbm, idx_hbm, out_hbm, table_v, idx_v, out_v,
          semi0, semi1, semo0, semo1):
    wid = lax.axis_index("s") * 2 + lax.axis_index("c")
    rg = wid >> 2
    hg = wid & 3

    pltpu.sync_copy(table_hbm.at[pl.ds(hg * _HPW * _V, _HPW * _V)], table_v)

    iota = jax.lax.iota(jnp.int32, _L)
    zeros = jnp.zeros((_L,), jnp.int32)
    for b in range(2):
        idx_v[pl.ds(b * _IDXBUF + _RPB * _N, _L)] = zeros
        idx_v[pl.ds(b * _IDXBUF + _RPB * _N + _L, _L)] = zeros

    semi = (semi0, semi1)
    semo = (semo0, semo1)

    def idx_off(s):
        return pl.multiple_of((rg * 128 + 8 * jnp.minimum(s, _STEPS - 1)) * _N, 8)

    def start_idx(s, p):
        pltpu.async_copy(
            idx_hbm.at[pl.ds(idx_off(s), _RPB * _N)],
            idx_v.at[pl.ds(p * _IDXBUF, _RPB * _N)], semi[p])

    def wait_idx(p):
        pltpu.make_async_copy(
            idx_hbm.at[pl.ds(0, _RPB * _N)],
            idx_v.at[pl.ds(p * _IDXBUF, _RPB * _N)], semi[p]).wait()

    def drain_out(p):
        for _ in range(_HPW):
            pltpu.make_async_copy(
                out_v.at[pl.ds(0, _RPB * _N)],
                out_hbm.at[0, pl.ds(0, _RPB * _N)], semo[p]).wait()

    def step(t, s, p):
        start_idx(s + 1, 1 - p)
        wait_idx(p)
        @pl.when(t >= 1)
        def _():
            drain_out(p)

        ibase = p * _IDXBUF
        obase = p * _OBUF

        for k in range(_RPB):
            koff = k * _N

            @plsc.parallel_loop(0, _GROUPS, unroll=2)
            def group(g):
                off = koff + g * _L
                idx16 = plsc.load_gather(idx_v, [iota + (ibase + off)])
                for hl in range(_HPW):
                    vals = plsc.load_gather(table_v, [idx16 + hl * _V])
                    plsc.store_scatter(
                        out_v, [iota + (obase + hl * _OBLK + off)], vals)

        row0 = rg * 128 + 8 * s
        for hl in range(_HPW):
            pltpu.async_copy(
                out_v.at[pl.ds(obase + hl * _OBLK, _RPB * _N)],
                out_hbm.at[hg * _HPW + hl,
                           pl.ds(pl.multiple_of(row0 * _N, 8), _RPB * _N)],
                semo[p])

    start_idx(0, 0)

    def pair(t, _):
        step(t, 2 * t, 0)
        step(t, 2 * t + 1, 1)
        return _

    lax.fori_loop(0, _STEPS // 2, pair, None)

    drain_out(0)
    drain_out(1)
    wait_idx(0)

    @pl.when(rg == 0)
    def _():
        idx_v[pl.ds(_N - 1, _L)] = zeros
        pltpu.sync_copy(idx_hbm.at[pl.ds(1024 * _N, _N)],
                        idx_v.at[pl.ds(0, _N)])

        @plsc.parallel_loop(0, _GROUPS, unroll=2)
        def group(g):
            off = g * _L
            idx16 = plsc.load_gather(idx_v, [iota + off])
            for hl in range(_HPW):
                vals = plsc.load_gather(table_v, [idx16 + hl * _V])
                plsc.store_scatter(out_v, [iota + (hl * _OBLK + off)], vals)

        for hl in range(_HPW):
            pltpu.sync_copy(
                out_v.at[pl.ds(hl * _OBLK, _N)],
                out_hbm.at[hg * _HPW + hl, pl.ds(1024 * _N, _N)])


@jax.jit
def _run(table_t_flat, idx_flat):
    mesh = plsc.VectorSubcoreMesh(core_axis_name="c", subcore_axis_name="s")
    f = pl.kernel(
        _body,
        mesh=mesh,
        out_type=jax.ShapeDtypeStruct((_H, _N2), jnp.float32),
        scratch_types=[
            pltpu.VMEM((_HPW * _V,), jnp.float32),
            pltpu.VMEM((2 * _IDXBUF,), jnp.int32),
            pltpu.VMEM((2 * _OBUF,), jnp.float32),
            pltpu.SemaphoreType.DMA,
            pltpu.SemaphoreType.DMA,
            pltpu.SemaphoreType.DMA,
            pltpu.SemaphoreType.DMA,
        ],
        compiler_params=pltpu.CompilerParams(
            needs_layout_passes=False, use_tc_tiling_on_sc=False
        ),
    )
    return f(table_t_flat, idx_flat)


def kernel(relative_position_bias_table, relative_position_index):
    table_t_flat = relative_position_bias_table.T.reshape(-1)
    idx_flat = relative_position_index.reshape(-1).astype(jnp.int32)
    return _run(table_t_flat, idx_flat).reshape(_H, _N, _N)


# speedup vs baseline: 7.2477x; 7.2477x over previous
import jax
import jax.numpy as jnp
from jax import lax
from jax.experimental import pallas as pl
from jax.experimental.pallas import tpu as pltpu
from jax.experimental.pallas import tpu_sc as plsc

_N = 1025
_H = 16
_V = 3972
_L = 16
_GROUPS = (_N + _L - 1) // _L
_NRG = 8
_NHG = 4
_HPW = _H // _NHG
_RPB = 8
_STEPS = 1024 // (_NRG * _RPB)


def _splat(v):
    return jnp.full((_L,), v, jnp.int32)


def _body(table_hbm, idx_hbm, out_hbm, table_v, idx_v, out_v,
          semi0, semi1, semo0, semo1):
    wid = lax.axis_index("s") * 2 + lax.axis_index("c")
    rg = wid >> 2
    hg = wid & 3

    pltpu.sync_copy(table_hbm.at[pl.ds(hg * _HPW, _HPW)], table_v)

    iota = jax.lax.iota(jnp.int32, _L)

    semi = (semi0, semi1)
    semo = (semo0, semo1)

    def row0_of(s):
        return rg * 128 + 8 * s

    def start_idx(s, p):
        r0 = pl.multiple_of(row0_of(jnp.minimum(s, _STEPS - 1)), 8)
        pltpu.async_copy(idx_hbm.at[pl.ds(r0, _RPB)], idx_v.at[p], semi[p])

    def wait_idx(p):
        pltpu.make_async_copy(
            idx_hbm.at[pl.ds(0, _RPB)], idx_v.at[p], semi[p]).wait()

    def drain_out(p):
        for _ in range(_HPW):
            pltpu.make_async_copy(
                out_v.at[0, 0, pl.ds(0, _RPB)],
                out_hbm.at[0, pl.ds(0, _RPB)], semo[p]).wait()

    def gather_row(p, k, dst_b, dst_k):
        @plsc.parallel_loop(0, _GROUPS, unroll=2)
        def group(g):
            col = iota + g * _L
            idx16 = plsc.load_gather(idx_v, [_splat(p), _splat(k), col])
            idx16 = jnp.bitwise_and(idx16, 4095)
            for hl in range(_HPW):
                vals = plsc.load_gather(table_v, [_splat(hl), idx16])
                plsc.store_scatter(
                    out_v, [_splat(dst_b), _splat(hl), _splat(dst_k), col],
                    vals)

    def step(t, s, p):
        start_idx(s + 1, 1 - p)
        wait_idx(p)
        @pl.when(t >= 1)
        def _():
            drain_out(p)

        for k in range(_RPB):
            gather_row(p, k, p, k)

        r0 = pl.multiple_of(row0_of(s), 8)
        for hl in range(_HPW):
            pltpu.async_copy(
                out_v.at[p, hl, pl.ds(0, _RPB)],
                out_hbm.at[hg * _HPW + hl, pl.ds(r0, _RPB)],
                semo[p])

    start_idx(0, 0)

    def pair(t, _):
        step(t, 2 * t, 0)
        step(t, 2 * t + 1, 1)
        return _

    lax.fori_loop(0, _STEPS // 2, pair, None)

    drain_out(0)
    drain_out(1)
    wait_idx(0)

    @pl.when(rg == 0)
    def _():
        pltpu.sync_copy(idx_hbm.at[1024], idx_v.at[0, 0])
        gather_row(0, 0, 0, 0)
        for hl in range(_HPW):
            pltpu.sync_copy(out_v.at[0, hl, 0], out_hbm.at[hg * _HPW + hl, 1024])


@jax.jit
def _run(table_t, idx2d):
    mesh = plsc.VectorSubcoreMesh(core_axis_name="c", subcore_axis_name="s")
    f = pl.kernel(
        _body,
        mesh=mesh,
        out_type=jax.ShapeDtypeStruct((_H, _N, _N), jnp.float32),
        scratch_types=[
            pltpu.VMEM((_HPW, _V), jnp.float32),
            pltpu.VMEM((2, _RPB, _N), jnp.int32),
            pltpu.VMEM((2, _HPW, _RPB + 1, _N), jnp.float32),
            pltpu.SemaphoreType.DMA,
            pltpu.SemaphoreType.DMA,
            pltpu.SemaphoreType.DMA,
            pltpu.SemaphoreType.DMA,
        ],
        compiler_params=pltpu.CompilerParams(
            needs_layout_passes=False, use_tc_tiling_on_sc=False
        ),
    )
    return f(table_t, idx2d)


def kernel(relative_position_bias_table, relative_position_index):
    table_t = relative_position_bias_table.T
    idx2d = relative_position_index.reshape(_N, _N).astype(jnp.int32)
    return _run(table_t, idx2d)
